# baseline (speedup 1.0000x reference)
import functools

import jax
import jax.numpy as jnp
from jax import lax
from jax.experimental import pallas as pl
from jax.experimental.pallas import tpu as pltpu
from jax.experimental.pallas import tpu_sc as plsc

N = 10000
E = 320000
D = 128
NC = 2
NS = 16
NW = NC * NS
EPW = 10240
EPAD = EPW * NW
B = 80
NCHUNK = EPW // B
CPB = 32
NBLK = NCHUNK // CPB
RPT = N // NS
NSUM = 10240
SPT = NSUM // NS


def _prep_body(h_ref, w_ref, asrc_ref, adst_ref, wh_ref, s_ref, d_ref, ub_ref):
    wh = jnp.dot(h_ref[...], w_ref[...], preferred_element_type=jnp.float32)
    s = jnp.dot(wh, asrc_ref[...], preferred_element_type=jnp.float32)
    d = jnp.dot(wh, adst_ref[...], preferred_element_type=jnp.float32)
    wh_ref[...] = wh
    s_ref[...] = s
    d_ref[...] = d
    ub0 = jnp.max(s) + jnp.max(d)
    ub = jnp.where(ub0 > 0, ub0, 0.01 * ub0)
    ub_ref[...] = jnp.full((8, 128), ub, jnp.float32)


_prep = pl.pallas_call(
    _prep_body,
    out_shape=[
        jax.ShapeDtypeStruct((N, D), jnp.float32),
        jax.ShapeDtypeStruct((N, 1), jnp.float32),
        jax.ShapeDtypeStruct((N, 1), jnp.float32),
        jax.ShapeDtypeStruct((8, 128), jnp.float32),
    ],
)


NB = 2


def _sc_body(src_hbm, dst_hbm, s_hbm, d_hbm, ub_hbm, wh_hbm,
             acc_out, sums_out, qmax_out,
             s_v, d_v, zsum, ub_v, qst,
             si_blk, di_blk, si0, si1, di0, di1, q0, q1, r0, r1,
             g0, g1, rs0, rs1, ss0, ss1,
             acc_sh, sums_sh):
    si = [si0, si1]
    di = [di0, di1]
    qv = [q0, q1]
    rows = [r0, r1]
    gsem = [g0, g1]
    rsem = [rs0, rs1]
    ssem = [ss0, ss1]

    c = lax.axis_index("c")
    sid = lax.axis_index("s")
    wid = c * NS + sid

    pltpu.sync_copy(s_hbm, s_v)
    pltpu.sync_copy(d_hbm, d_v)
    pltpu.sync_copy(ub_hbm, ub_v)

    def zfill(i, carry):
        for cc in range(D // 16):
            r0[i, pl.ds(cc * 16, 16)] = jnp.zeros((16,), jnp.float32)
        return carry
    lax.fori_loop(0, B, zfill, 0)

    def zsfill(i, carry):
        zsum[pl.ds(i * 16, 16)] = jnp.zeros((16,), jnp.float32)
        return carry
    lax.fori_loop(0, SPT // 16, zsfill, 0)

    row0 = sid * RPT
    for k in range(RPT // B):
        pltpu.sync_copy(r0, acc_sh.at[pl.ds(row0 + k * B, B)])
    rem = RPT - (RPT // B) * B
    if rem:
        pltpu.sync_copy(r0.at[pl.ds(0, rem)],
                        acc_sh.at[pl.ds(row0 + (RPT // B) * B, rem)])
    pltpu.sync_copy(zsum, sums_sh.at[pl.ds(sid * SPT, SPT)])
    plsc.subcore_barrier()

    ub = ub_v[...]
    ebase = wid * EPW

    def drain(b):
        pltpu.make_async_copy(rows[b], acc_sh.at[di[b]], rsem[b]).wait()
        pltpu.make_async_copy(qv[b], sums_sh.at[di[b]], ssem[b]).wait()

    def issue(ci, b):
        for k in range(B // 16):
            si[b][pl.ds(k * 16, 16)] = si_blk[pl.ds(ci * B + k * 16, 16)]
            di[b][pl.ds(k * 16, 16)] = di_blk[pl.ds(ci * B + k * 16, 16)]
        pltpu.async_copy(wh_hbm.at[si[b]], rows[b], gsem[b])

    def process(gci, b, qmax):
        def qbody(i, qm):
            sv = si[b][pl.ds(i * 16, 16)]
            dv = di[b][pl.ds(i * 16, 16)]
            e = plsc.load_gather(s_v, [sv]) + plsc.load_gather(d_v, [dv])
            e = jnp.where(e > 0, e, 0.01 * e)
            q = jnp.exp(e - ub)
            gid = ebase + gci * B + i * 16 + jnp.arange(16, dtype=jnp.int32)
            q = jnp.where(gid < E, q, 0.0)
            qv[b][pl.ds(i * 16, 16)] = q
            return jnp.maximum(qm, q)
        qmax = lax.fori_loop(0, B // 16, qbody, qmax)

        pltpu.make_async_copy(wh_hbm.at[si[b]], rows[b], gsem[b]).wait()

        @plsc.parallel_loop(0, B, 1, unroll=4)
        def sbody(j):
            qb = plsc.load_gather(qv[b], [jnp.full((16,), j, jnp.int32)])
            for cc in range(D // 16):
                rows[b][j, pl.ds(cc * 16, 16)] = (
                    rows[b][j, pl.ds(cc * 16, 16)] * qb)

        pltpu.async_copy(rows[b], acc_sh.at[di[b]], rsem[b], add=True)
        pltpu.async_copy(qv[b], sums_sh.at[di[b]], ssem[b], add=True)
        return qmax

    qmax = jnp.zeros((16,), jnp.float32)
    for blk in range(NBLK):
        bbase = ebase + blk * CPB * B
        pltpu.sync_copy(src_hbm.at[pl.ds(bbase, CPB * B)], si_blk)
        pltpu.sync_copy(dst_hbm.at[pl.ds(bbase, CPB * B)], di_blk)
        issue(0, 0)

        def pair_body(g, qm, _blk=blk):
            gc0 = _blk * CPB + 2 * g
            @pl.when(g > 0)
            def _():
                drain(1)
            issue(2 * g + 1, 1)
            qm = process(gc0, 0, qm)
            qm = process(gc0 + 1, 1, qm)
            drain(0)
            @pl.when(g < CPB // 2 - 1)
            def _():
                issue(2 * g + 2, 0)
            return qm

        qmax = lax.fori_loop(0, CPB // 2, pair_body, qmax)
        drain(1)

    plsc.subcore_barrier()
    pltpu.sync_copy(acc_sh.at[pl.ds(row0, RPT)],
                    acc_out.at[c, pl.ds(row0, RPT)])
    pltpu.sync_copy(sums_sh.at[pl.ds(sid * SPT, SPT)],
                    sums_out.at[c, pl.ds(sid * SPT, SPT)])
    qst[0, :] = qmax
    pltpu.sync_copy(qst, qmax_out.at[pl.ds(wid, 1)])


@functools.lru_cache(maxsize=None)
def _make_sc_edges():
  return functools.partial(
    pl.kernel,
    out_type=[
        jax.ShapeDtypeStruct((NC, N, D), jnp.float32),
        jax.ShapeDtypeStruct((NC, NSUM), jnp.float32),
        jax.ShapeDtypeStruct((NW, 16), jnp.float32),
    ],
    mesh=plsc.VectorSubcoreMesh(core_axis_name="c", subcore_axis_name="s",
                                num_cores=NC, num_subcores=NS),
    compiler_params=pltpu.CompilerParams(use_tc_tiling_on_sc=False,
                                         needs_layout_passes=False),
    scratch_types=(
        [
            pltpu.VMEM((N,), jnp.float32),
            pltpu.VMEM((N,), jnp.float32),
            pltpu.VMEM((SPT,), jnp.float32),
            pltpu.VMEM((16,), jnp.float32),
            pltpu.VMEM((1, 16), jnp.float32),
        ]
        + [pltpu.VMEM((CPB * B,), jnp.int32) for _ in range(2)]
        + [pltpu.VMEM((B,), jnp.int32) for _ in range(2 * NB)]
        + [pltpu.VMEM((B,), jnp.float32) for _ in range(NB)]
        + [pltpu.VMEM((B, D), jnp.float32) for _ in range(NB)]
        + [pltpu.SemaphoreType.DMA for _ in range(3 * NB)]
        + [
            pltpu.VMEM_SHARED((N, D), jnp.float32),
            pltpu.VMEM_SHARED((NSUM,), jnp.float32),
        ]
    ),
  )(_sc_body)


def _fin_body(acc_ref, sums_ref, qmax_ref, out_ref):
    acc = acc_ref[0] + acc_ref[1]
    sums = sums_ref[0, :N] + sums_ref[1, :N]
    qmax = jnp.max(qmax_ref[...])
    denom = jnp.reshape(sums, (N, 1)) + 1e-9 * qmax
    res = acc / denom
    out_ref[...] = jnp.where(res > 0, res, jnp.exp(res) - 1.0)


_fin = pl.pallas_call(
    _fin_body,
    out_shape=jax.ShapeDtypeStruct((N, D), jnp.float32),
)


def kernel(h, edge_index, W, a_src, a_dst):
    wh, s, d, ubf = _prep(h, W, a_src, a_dst)
    pad = jnp.zeros((2, EPAD - E), jnp.int32)
    ei = jnp.concatenate([edge_index, pad], axis=1)
    src = ei[0]
    dst = ei[1]
    s1 = jnp.reshape(s, (N,))
    d1 = jnp.reshape(d, (N,))
    ub16 = ubf[0, :16]
    acc, sums, qmaxp = _make_sc_edges()(src, dst, s1, d1, ub16, wh)
    return _fin(acc, sums, qmaxp)


# speedup vs baseline: 2.4901x; 2.4901x over previous
import functools

import jax
import jax.numpy as jnp
from jax import lax
from jax.experimental import pallas as pl
from jax.experimental.pallas import tpu as pltpu
from jax.experimental.pallas import tpu_sc as plsc

N = 10000
E = 320000
D = 128
NC = 2
NS = 16
NW = NC * NS
EPW = 10240
EPAD = EPW * NW
B = 80
NCHUNK = EPW // B
CPB = 32
NBLK = NCHUNK // CPB
RPT = N // NS
NSUM = 10240
SPT = NSUM // NS


def _prep_body(h_ref, w_ref, asrc_ref, adst_ref, wh_ref, s_ref, d_ref, ub_ref):
    wh = jnp.dot(h_ref[...], w_ref[...], preferred_element_type=jnp.float32)
    s = jnp.dot(wh, asrc_ref[...], preferred_element_type=jnp.float32)
    d = jnp.dot(wh, adst_ref[...], preferred_element_type=jnp.float32)
    wh_ref[...] = wh
    s_ref[...] = s
    d_ref[...] = d
    ub0 = jnp.max(s) + jnp.max(d)
    ub = jnp.where(ub0 > 0, ub0, 0.01 * ub0)
    ub_ref[...] = jnp.full((8, 128), ub, jnp.float32)


_prep = pl.pallas_call(
    _prep_body,
    out_shape=[
        jax.ShapeDtypeStruct((N, D), jnp.float32),
        jax.ShapeDtypeStruct((N, 1), jnp.float32),
        jax.ShapeDtypeStruct((N, 1), jnp.float32),
        jax.ShapeDtypeStruct((8, 128), jnp.float32),
    ],
)


NB = 2


def _sc_body(src_hbm, dst_hbm, s_hbm, d_hbm, ub_hbm, wh_hbm,
             acc_out, sums_out, qmax_out,
             s_v, d_v, zsum, ub_v, qst,
             si_blk, di_blk, si0, si1, di0, di1, q0, q1, r0, r1,
             g0, g1, rs0, rs1, ss0, ss1,
             acc_sh, sums_sh):
    si = [si0, si1]
    di = [di0, di1]
    qv = [q0, q1]
    rows = [r0, r1]
    gsem = [g0, g1]
    rsem = [rs0, rs1]
    ssem = [ss0, ss1]

    c = lax.axis_index("c")
    sid = lax.axis_index("s")
    wid = c * NS + sid

    pltpu.sync_copy(s_hbm, s_v)
    pltpu.sync_copy(d_hbm, d_v)
    pltpu.sync_copy(ub_hbm, ub_v)

    def zfill(i, carry):
        for cc in range(D // 16):
            r0[i, pl.ds(cc * 16, 16)] = jnp.zeros((16,), jnp.float32)
        return carry
    lax.fori_loop(0, B, zfill, 0)

    def zsfill(i, carry):
        zsum[pl.ds(i * 16, 16)] = jnp.zeros((16,), jnp.float32)
        return carry
    lax.fori_loop(0, SPT // 16, zsfill, 0)

    row0 = sid * RPT
    for k in range(RPT // B):
        pltpu.sync_copy(r0, acc_sh.at[pl.ds(row0 + k * B, B)])
    rem = RPT - (RPT // B) * B
    if rem:
        pltpu.sync_copy(r0.at[pl.ds(0, rem)],
                        acc_sh.at[pl.ds(row0 + (RPT // B) * B, rem)])
    pltpu.sync_copy(zsum, sums_sh.at[pl.ds(sid * SPT, SPT)])
    plsc.subcore_barrier()

    ub = ub_v[...]
    ebase = wid * EPW

    def drain(b):
        pltpu.make_async_copy(rows[b], acc_sh.at[di[b]], rsem[b]).wait()
        pltpu.make_async_copy(qv[b], sums_sh.at[di[b]], ssem[b]).wait()

    def issue(ci, b):
        for k in range(B // 16):
            si[b][pl.ds(k * 16, 16)] = si_blk[pl.ds(ci * B + k * 16, 16)]
            di[b][pl.ds(k * 16, 16)] = di_blk[pl.ds(ci * B + k * 16, 16)]
        pltpu.async_copy(wh_hbm.at[si[b]], rows[b], gsem[b])

    def process(gci, b, qmax):
        def qbody(i, qm):
            sv = si[b][pl.ds(i * 16, 16)]
            dv = di[b][pl.ds(i * 16, 16)]
            e = plsc.load_gather(s_v, [sv]) + plsc.load_gather(d_v, [dv])
            e = jnp.where(e > 0, e, 0.01 * e)
            q = jnp.exp(e - ub)
            gid = ebase + gci * B + i * 16 + jnp.arange(16, dtype=jnp.int32)
            q = jnp.where(gid < E, q, 0.0)
            qv[b][pl.ds(i * 16, 16)] = q
            return jnp.maximum(qm, q)
        qmax = lax.fori_loop(0, B // 16, qbody, qmax)

        pltpu.make_async_copy(wh_hbm.at[si[b]], rows[b], gsem[b]).wait()

        @plsc.parallel_loop(0, B, 1, unroll=4)
        def sbody(j):
            qb = plsc.load_gather(qv[b], [jnp.full((16,), j, jnp.int32)])
            for cc in range(D // 16):
                rows[b][j, pl.ds(cc * 16, 16)] = (
                    rows[b][j, pl.ds(cc * 16, 16)] * qb)

        pltpu.async_copy(rows[b], acc_sh.at[di[b]], rsem[b], add=True)
        pltpu.async_copy(qv[b], sums_sh.at[di[b]], ssem[b], add=True)
        return qmax

    qmax = jnp.zeros((16,), jnp.float32)
    for blk in range(NBLK):
        bbase = ebase + blk * CPB * B
        pltpu.sync_copy(src_hbm.at[pl.ds(bbase, CPB * B)], si_blk)
        pltpu.sync_copy(dst_hbm.at[pl.ds(bbase, CPB * B)], di_blk)
        issue(0, 0)

        def pair_body(g, qm, _blk=blk):
            gc0 = _blk * CPB + 2 * g
            @pl.when(g > 0)
            def _():
                drain(1)
            issue(2 * g + 1, 1)
            qm = process(gc0, 0, qm)
            qm = process(gc0 + 1, 1, qm)
            drain(0)
            @pl.when(g < CPB // 2 - 1)
            def _():
                issue(2 * g + 2, 0)
            return qm

        qmax = lax.fori_loop(0, CPB // 2, pair_body, qmax)
        drain(1)

    plsc.subcore_barrier()
    pltpu.sync_copy(acc_sh.at[pl.ds(row0, RPT)],
                    acc_out.at[c, pl.ds(row0, RPT)])
    pltpu.sync_copy(sums_sh.at[pl.ds(sid * SPT, SPT)],
                    sums_out.at[c, pl.ds(sid * SPT, SPT)])
    qst[0, :] = qmax
    pltpu.sync_copy(qst, qmax_out.at[pl.ds(wid, 1)])


@functools.lru_cache(maxsize=None)
def _make_sc_edges():
  return functools.partial(
    pl.kernel,
    out_type=[
        jax.ShapeDtypeStruct((NC, N, D), jnp.float32),
        jax.ShapeDtypeStruct((NC, NSUM), jnp.float32),
        jax.ShapeDtypeStruct((NW, 16), jnp.float32),
    ],
    mesh=plsc.VectorSubcoreMesh(core_axis_name="c", subcore_axis_name="s",
                                num_cores=NC, num_subcores=NS),
    compiler_params=pltpu.CompilerParams(use_tc_tiling_on_sc=False,
                                         needs_layout_passes=False),
    scratch_types=(
        [
            pltpu.VMEM((N,), jnp.float32),
            pltpu.VMEM((N,), jnp.float32),
            pltpu.VMEM((SPT,), jnp.float32),
            pltpu.VMEM((16,), jnp.float32),
            pltpu.VMEM((1, 16), jnp.float32),
        ]
        + [pltpu.VMEM((CPB * B,), jnp.int32) for _ in range(2)]
        + [pltpu.VMEM((B,), jnp.int32) for _ in range(2 * NB)]
        + [pltpu.VMEM((B,), jnp.float32) for _ in range(NB)]
        + [pltpu.VMEM((B, D), jnp.float32) for _ in range(NB)]
        + [pltpu.SemaphoreType.DMA for _ in range(3 * NB)]
        + [
            pltpu.VMEM_SHARED((N, D), jnp.float32),
            pltpu.VMEM_SHARED((NSUM,), jnp.float32),
        ]
    ),
  )(_sc_body)


def _fin_body(acc_ref, sums_ref, qmax_ref, out_ref):
    acc = acc_ref[0] + acc_ref[1]
    sums = sums_ref[0, :N] + sums_ref[1, :N]
    qmax = jnp.max(qmax_ref[...])
    denom = jnp.reshape(sums, (N, 1)) + 1e-9 * qmax
    res = acc / denom
    out_ref[...] = jnp.where(res > 0, res, jnp.exp(res) - 1.0)


_fin = pl.pallas_call(
    _fin_body,
    out_shape=jax.ShapeDtypeStruct((N, D), jnp.float32),
)


def kernel(h, edge_index, W, a_src, a_dst):
    wh, s, d, ubf = _prep(h, W, a_src, a_dst)
    idx_pad = jnp.arange(EPAD - E, dtype=jnp.int32) % N
    ei = jnp.concatenate([edge_index, jnp.stack([idx_pad, idx_pad])], axis=1)
    src = ei[0]
    dst = ei[1]
    s1 = jnp.reshape(s, (N,))
    d1 = jnp.reshape(d, (N,))
    ub16 = ubf[0, :16]
    acc, sums, qmaxp = _make_sc_edges()(src, dst, s1, d1, ub16, wh)
    return _fin(acc, sums, qmaxp)
